# baseline (speedup 1.0000x reference)
import functools

import jax
import jax.numpy as jnp
from jax import lax
from jax.experimental import pallas as pl
from jax.experimental.pallas import tpu as pltpu
from jax.experimental.pallas import tpu_sc as plsc

NUM_CORES = 2
NUM_SUBCORES = 16
NUM_WORKERS = NUM_CORES * NUM_SUBCORES

CHUNK = 128


def _gather(idx, weight, total, D, n_chunks):
    per_worker = total // NUM_WORKERS
    n_pairs = n_chunks // 2
    mesh = plsc.VectorSubcoreMesh(core_axis_name="c", subcore_axis_name="s")

    @functools.partial(
        pl.kernel,
        mesh=mesh,
        out_type=jax.ShapeDtypeStruct((total, D), jnp.float32),
        scratch_types=[
            pltpu.VMEM((n_chunks, CHUNK), jnp.int32),
            pltpu.VMEM((CHUNK, D), jnp.float32),
            pltpu.VMEM((CHUNK, D), jnp.float32),
            pltpu.SemaphoreType.DMA,
            pltpu.SemaphoreType.DMA,
        ],
        compiler_params=pltpu.CompilerParams(use_tc_tiling_on_sc=False),
    )
    def gather_kernel(idx_hbm, table_hbm, out_hbm, idx_v, rows_a, rows_b, sem_a, sem_b):
        wid = lax.axis_index("s") * NUM_CORES + lax.axis_index("c")
        base = wid * per_worker
        pltpu.sync_copy(idx_hbm.at[wid], idx_v)

        pltpu.async_copy(table_hbm.at[idx_v.at[0]], rows_a, sem_a)

        def body(i, carry):
            g = 2 * i
            pltpu.async_copy(table_hbm.at[idx_v.at[g + 1]], rows_b, sem_b)
            pltpu.make_async_copy(table_hbm.at[idx_v.at[g]], rows_a, sem_a).wait()
            pltpu.sync_copy(rows_a, out_hbm.at[pl.ds(base + g * CHUNK, CHUNK)])

            @pl.when(i < n_pairs - 1)
            def _():
                pltpu.async_copy(table_hbm.at[idx_v.at[g + 2]], rows_a, sem_a)

            pltpu.make_async_copy(table_hbm.at[idx_v.at[g + 1]], rows_b, sem_b).wait()
            pltpu.sync_copy(rows_b, out_hbm.at[pl.ds(base + (g + 1) * CHUNK, CHUNK)])
            return carry

        lax.fori_loop(0, n_pairs, body, 0)

    return gather_kernel(idx, weight)


def _transpose_tc(pairs, B, S, D):
    P = S // 2
    nblk = B // 128

    def body(in_ref, out_ref):
        x = in_ref[0]
        out_ref[0] = x[:, :D].T
        out_ref[1] = x[:, D:].T

    return pl.pallas_call(
        body,
        grid=(P, nblk),
        in_specs=[pl.BlockSpec((1, 128, 128), lambda p, b: (p, b, 0))],
        out_specs=pl.BlockSpec((2, D, 128), lambda p, b: (p, 0, b)),
        out_shape=jax.ShapeDtypeStruct((S, D, B), jnp.float32),
    )(pairs)


def kernel(token_ids, weight):
    B, S = token_ids.shape
    V, D = weight.shape
    P = S // 2
    total = B * S
    n_chunks = (total // NUM_WORKERS) // CHUNK

    idx3 = token_ids.reshape(B, P, 2).transpose(1, 0, 2).astype(jnp.int32)
    idx = idx3.reshape(NUM_WORKERS, n_chunks, CHUNK)

    rows = _gather(idx, weight, total, D, n_chunks)
    pairs = rows.reshape(P, B, 2 * D)
    outc = _transpose_tc(pairs, B, S, D)
    return jnp.transpose(outc, (2, 0, 1))
